# baseline (speedup 1.0000x reference)
import functools

import jax
import jax.numpy as jnp
from jax import lax
from jax.experimental import pallas as pl
from jax.experimental.pallas import tpu as pltpu
from jax.experimental.pallas import tpu_sc as plsc

T = 2048
E = 16
D = 768
H = 1536
B = 128
NB = 32
TP = NB * B

NW = 32
TPW = T // NW

_SC_MESH = plsc.VectorSubcoreMesh(
    core_axis_name="c", subcore_axis_name="s", num_cores=2, num_subcores=16)


def _vgather(vec, idx):
    return vec.at[idx].get(mode="promise_in_bounds")


def _router_body(x_ref, wr_ref, idx_ref, loss_ref):
    x = x_ref[...]
    wr = wr_ref[...]
    logits = lax.dot_general(x, wr, (((1,), (1,)), ((), ())),
                             preferred_element_type=jnp.float32)
    m = jnp.max(logits, axis=-1, keepdims=True)
    eiota = lax.broadcasted_iota(jnp.int32, (T, E), 1)
    idx = jnp.min(jnp.where(logits >= m, eiota, E), axis=-1, keepdims=True)
    p = jnp.exp(logits - m)
    probs = p / jnp.sum(p, axis=-1, keepdims=True)
    onehot = (eiota == idx).astype(jnp.float32)
    su = jnp.sum(probs, axis=0)
    sc = jnp.sum(onehot, axis=0)
    loss = (float(E) / (float(T) * float(T))) * jnp.sum(su * sc)
    loss_ref[...] = loss.reshape(1, 1)
    idx_ref[...] = idx


def _router(x, wr):
    return pl.pallas_call(
        _router_body,
        out_shape=[
            jax.ShapeDtypeStruct((T, 1), jnp.int32),
            jax.ShapeDtypeStruct((1, 1), jnp.float32),
        ],
    )(x, wr)


def _count_body(idx_hbm, lp_hbm, hist_hbm, e_v, lp_v, cnt_v):
    cid = lax.axis_index("c")
    sid = lax.axis_index("s")
    w = sid * 2 + cid
    base_t = w * TPW

    pltpu.sync_copy(idx_hbm.at[pl.ds(base_t, TPW)], e_v)
    liota = lax.iota(jnp.int32, 16)
    ones = jnp.full((16,), 1, jnp.int32)
    zeros = jnp.zeros((16,), jnp.int32)

    cnt = zeros
    for c in range(TPW // 16):
        e = e_v[pl.ds(c * 16, 16)]
        prior = _vgather(cnt, e)
        rank = zeros
        for j in range(16):
            bj = _vgather(e, jnp.full((16,), j, jnp.int32))
            after_j = jnp.where(liota > j, ones, zeros)
            rank = rank + jnp.where(e == bj, after_j, zeros)
            cnt = cnt + jnp.where(liota == bj, ones, zeros)
        lp_v[pl.ds(c * 16, 16)] = prior + rank

    cnt_v[...] = cnt
    pltpu.sync_copy(lp_v, lp_hbm.at[pl.ds(base_t, TPW)])
    pltpu.sync_copy(cnt_v, hist_hbm.at[w])


@functools.partial(
    pl.kernel,
    out_type=[
        jax.ShapeDtypeStruct((T,), jnp.int32),
        jax.ShapeDtypeStruct((NW, 16), jnp.int32),
    ],
    mesh=_SC_MESH,
    scratch_types=[
        pltpu.VMEM((TPW,), jnp.int32),
        pltpu.VMEM((TPW,), jnp.int32),
        pltpu.VMEM((16,), jnp.int32),
    ],
)
def _count(idx_hbm, lp_hbm, hist_hbm, *scratch):
    _count_body(idx_hbm, lp_hbm, hist_hbm, *scratch)


def _dispatch_body(idx_hbm, lp_hbm, histin_hbm, x_hbm, pos_hbm, meta_hbm,
                   xs_hbm, e_v, pos_v, hist_v, meta_v, xrows_v, sem):
    cid = lax.axis_index("c")
    sid = lax.axis_index("s")
    w = sid * 2 + cid
    base_t = w * TPW

    pltpu.sync_copy(histin_hbm, hist_v)
    pltpu.sync_copy(idx_hbm.at[pl.ds(base_t, TPW)], e_v)
    pltpu.sync_copy(lp_hbm.at[pl.ds(base_t, TPW)], pos_v)

    liota = lax.iota(jnp.int32, 16)
    ones = jnp.full((16,), 1, jnp.int32)
    zeros = jnp.zeros((16,), jnp.int32)

    tot = zeros
    mine = zeros
    for wp in range(NW):
        h = hist_v[wp]
        mine = mine + jnp.where(wp < w, h, zeros)
        tot = tot + h

    nblk = lax.div(tot + (B - 1), jnp.full((16,), B, jnp.int32))
    cumn = nblk
    for sh in (1, 2, 4, 8):
        sf = _vgather(cumn, jnp.maximum(liota - sh, zeros))
        cumn = cumn + jnp.where(liota >= sh, sf, zeros)
    basev = (cumn - nblk) * B + mine

    for c in range(TPW // 16):
        e = e_v[pl.ds(c * 16, 16)]
        lp = pos_v[pl.ds(c * 16, 16)]
        pos_v[pl.ds(c * 16, 16)] = _vgather(basev, e) + lp

    pltpu.sync_copy(pos_v, pos_hbm.at[pl.ds(base_t, TPW)])

    pltpu.sync_copy(x_hbm.at[pl.ds(base_t, TPW)], xrows_v)
    pltpu.async_copy(xrows_v, xs_hbm.at[pos_v], sem).wait()

    @pl.when(w == 0)
    def _():
        nact = _vgather(cumn, jnp.full((16,), 15, jnp.int32))
        for half in range(2):
            bvec = liota + half * 16
            blk_e = zeros
            for ep in range(16):
                cb = _vgather(cumn, jnp.full((16,), ep, jnp.int32))
                blk_e = blk_e + jnp.where(bvec >= cb, ones, zeros)
            meta_v[pl.ds(half * 16, 16)] = jnp.minimum(blk_e, 15)
        meta_v[pl.ds(32, 16)] = nact
        pltpu.sync_copy(meta_v, meta_hbm)


@functools.partial(
    pl.kernel,
    out_type=[
        jax.ShapeDtypeStruct((T,), jnp.int32),
        jax.ShapeDtypeStruct((48,), jnp.int32),
        jax.ShapeDtypeStruct((TP, D), jnp.float32),
    ],
    mesh=_SC_MESH,
    scratch_types=[
        pltpu.VMEM((TPW,), jnp.int32),
        pltpu.VMEM((TPW,), jnp.int32),
        pltpu.VMEM((NW, 16), jnp.int32),
        pltpu.VMEM((48,), jnp.int32),
        pltpu.VMEM((TPW, D), jnp.float32),
        pltpu.SemaphoreType.DMA,
    ],
)
def _dispatch(idx_hbm, lp_hbm, histin_hbm, x_hbm, pos_hbm, meta_hbm, xs_hbm,
              *scratch):
    _dispatch_body(idx_hbm, lp_hbm, histin_hbm, x_hbm, pos_hbm, meta_hbm,
                   xs_hbm, *scratch)


def _ffn_body(meta_ref, xs_ref, w1_ref, b1_ref, w2_ref, b2_ref, out_ref):
    @pl.when(pl.program_id(0) < meta_ref[32])
    def _():
        x = xs_ref[...].astype(jnp.bfloat16)
        h = lax.dot_general(x, w1_ref[0], (((1,), (1,)), ((), ())),
                            preferred_element_type=jnp.float32)
        h = h + b1_ref[0]
        h = 0.5 * h * (1.0 + lax.erf(h * 0.7071067811865476))
        o = lax.dot_general(h.astype(jnp.bfloat16), w2_ref[0],
                            (((1,), (1,)), ((), ())),
                            preferred_element_type=jnp.float32)
        out_ref[...] = o + b2_ref[0]


def _ffn(meta, xs, w1, b1, w2, b2):
    grid_spec = pltpu.PrefetchScalarGridSpec(
        num_scalar_prefetch=1,
        grid=(NB,),
        in_specs=[
            pl.BlockSpec((B, D), lambda i, m: (i, 0)),
            pl.BlockSpec((1, H, D), lambda i, m: (m[i], 0, 0)),
            pl.BlockSpec((1, 1, H), lambda i, m: (m[i], 0, 0)),
            pl.BlockSpec((1, D, H), lambda i, m: (m[i], 0, 0)),
            pl.BlockSpec((1, 1, D), lambda i, m: (m[i], 0, 0)),
        ],
        out_specs=pl.BlockSpec((B, D), lambda i, m: (i, 0)),
    )
    return pl.pallas_call(
        _ffn_body,
        grid_spec=grid_spec,
        out_shape=jax.ShapeDtypeStruct((TP, D), jnp.float32),
    )(meta, xs, w1, b1, w2, b2)


def _combine_body(ys_hbm, pos_hbm, out_hbm, pos_v, rows_v, sem):
    cid = lax.axis_index("c")
    sid = lax.axis_index("s")
    w = sid * 2 + cid
    base_t = w * TPW
    pltpu.sync_copy(pos_hbm.at[pl.ds(base_t, TPW)], pos_v)
    pltpu.async_copy(ys_hbm.at[pos_v], rows_v, sem).wait()
    pltpu.sync_copy(rows_v, out_hbm.at[pl.ds(base_t, TPW)])


@functools.partial(
    pl.kernel,
    out_type=jax.ShapeDtypeStruct((T, D), jnp.float32),
    mesh=_SC_MESH,
    scratch_types=[
        pltpu.VMEM((TPW,), jnp.int32),
        pltpu.VMEM((TPW, D), jnp.float32),
        pltpu.SemaphoreType.DMA,
    ],
)
def _combine(ys_hbm, pos_hbm, out_hbm, pos_v, rows_v, sem):
    _combine_body(ys_hbm, pos_hbm, out_hbm, pos_v, rows_v, sem)


def kernel(x, Wr, W1, b1, W2, b2):
    idx2d, loss2d = _router(x, Wr)
    idx = idx2d.reshape(T)
    lp, hist = _count(idx)
    pos, meta, xs = _dispatch(idx, lp, hist, x)
    ys = _ffn(meta, xs, W1.astype(jnp.bfloat16), b1.reshape(E, 1, H),
              W2.astype(jnp.bfloat16), b2.reshape(E, 1, D))
    out = _combine(ys, pos)
    return out, loss2d[0, 0]


# speedup vs baseline: 1.3972x; 1.3972x over previous
import functools

import jax
import jax.numpy as jnp
from jax import lax
from jax.experimental import pallas as pl
from jax.experimental.pallas import tpu as pltpu
from jax.experimental.pallas import tpu_sc as plsc

T = 2048
E = 16
D = 768
H = 1536
B = 128
NB = 32
TP = NB * B

NW = 32
TPW = T // NW

_SC_MESH = plsc.VectorSubcoreMesh(
    core_axis_name="c", subcore_axis_name="s", num_cores=2, num_subcores=16)


def _vgather(vec, idx):
    return vec.at[idx].get(mode="promise_in_bounds")


def _router_body(x_ref, wr_ref, idx_ref, loss_ref):
    x = x_ref[...]
    wr = wr_ref[...]
    logits = lax.dot_general(x, wr, (((1,), (1,)), ((), ())),
                             preferred_element_type=jnp.float32)
    m = jnp.max(logits, axis=-1, keepdims=True)
    eiota = lax.broadcasted_iota(jnp.int32, (T, E), 1)
    idx = jnp.min(jnp.where(logits >= m, eiota, E), axis=-1, keepdims=True)
    p = jnp.exp(logits - m)
    probs = p / jnp.sum(p, axis=-1, keepdims=True)
    onehot = (eiota == idx).astype(jnp.float32)
    su = jnp.sum(probs, axis=0)
    sc = jnp.sum(onehot, axis=0)
    loss = (float(E) / (float(T) * float(T))) * jnp.sum(su * sc)
    loss_ref[...] = loss.reshape(1, 1)
    idx_ref[...] = idx


def _router(x, wr):
    return pl.pallas_call(
        _router_body,
        out_shape=[
            jax.ShapeDtypeStruct((T, 1), jnp.int32),
            jax.ShapeDtypeStruct((1, 1), jnp.float32),
        ],
    )(x, wr)


def _count_body(idx_hbm, lp_hbm, hist_hbm, e_v, lp_v, cnt_v):
    cid = lax.axis_index("c")
    sid = lax.axis_index("s")
    w = sid * 2 + cid
    base_t = w * TPW

    pltpu.sync_copy(idx_hbm.at[pl.ds(base_t, TPW)], e_v)
    liota = lax.iota(jnp.int32, 16)
    ones = jnp.full((16,), 1, jnp.int32)
    zeros = jnp.zeros((16,), jnp.int32)

    cnt = zeros
    for c in range(TPW // 16):
        e = e_v[pl.ds(c * 16, 16)]
        prior = _vgather(cnt, e)
        rank = zeros
        for j in range(16):
            bj = _vgather(e, jnp.full((16,), j, jnp.int32))
            after_j = jnp.where(liota > j, ones, zeros)
            rank = rank + jnp.where(e == bj, after_j, zeros)
            cnt = cnt + jnp.where(liota == bj, ones, zeros)
        lp_v[pl.ds(c * 16, 16)] = prior + rank

    cnt_v[...] = cnt
    pltpu.sync_copy(lp_v, lp_hbm.at[pl.ds(base_t, TPW)])
    pltpu.sync_copy(cnt_v, hist_hbm.at[w])


@functools.partial(
    pl.kernel,
    out_type=[
        jax.ShapeDtypeStruct((T,), jnp.int32),
        jax.ShapeDtypeStruct((NW, 16), jnp.int32),
    ],
    mesh=_SC_MESH,
    scratch_types=[
        pltpu.VMEM((TPW,), jnp.int32),
        pltpu.VMEM((TPW,), jnp.int32),
        pltpu.VMEM((16,), jnp.int32),
    ],
)
def _count(idx_hbm, lp_hbm, hist_hbm, *scratch):
    _count_body(idx_hbm, lp_hbm, hist_hbm, *scratch)


def _dispatch_body(idx_hbm, lp_hbm, histin_hbm, x_hbm, pos_hbm, meta_hbm,
                   xs_hbm, e_v, pos_v, hist_v, meta_v, xrows_v, sem):
    cid = lax.axis_index("c")
    sid = lax.axis_index("s")
    w = sid * 2 + cid
    base_t = w * TPW

    pltpu.sync_copy(histin_hbm, hist_v)
    pltpu.sync_copy(idx_hbm.at[pl.ds(base_t, TPW)], e_v)
    pltpu.sync_copy(lp_hbm.at[pl.ds(base_t, TPW)], pos_v)

    liota = lax.iota(jnp.int32, 16)
    ones = jnp.full((16,), 1, jnp.int32)
    zeros = jnp.zeros((16,), jnp.int32)

    tot = zeros
    mine = zeros
    for wp in range(NW):
        h = hist_v[wp]
        mine = mine + jnp.where(wp < w, h, zeros)
        tot = tot + h

    nblk = lax.div(tot + (B - 1), jnp.full((16,), B, jnp.int32))
    cumn = nblk
    for sh in (1, 2, 4, 8):
        sf = _vgather(cumn, jnp.maximum(liota - sh, zeros))
        cumn = cumn + jnp.where(liota >= sh, sf, zeros)
    basev = (cumn - nblk) * B + mine

    for c in range(TPW // 16):
        e = e_v[pl.ds(c * 16, 16)]
        lp = pos_v[pl.ds(c * 16, 16)]
        pos_v[pl.ds(c * 16, 16)] = _vgather(basev, e) + lp

    pltpu.sync_copy(pos_v, pos_hbm.at[pl.ds(base_t, TPW)])

    pltpu.sync_copy(x_hbm.at[pl.ds(base_t, TPW)], xrows_v)
    pltpu.async_copy(xrows_v, xs_hbm.at[pos_v], sem).wait()

    @pl.when(w == 0)
    def _():
        nact = _vgather(cumn, jnp.full((16,), 15, jnp.int32))
        for half in range(2):
            bvec = liota + half * 16
            blk_e = zeros
            for ep in range(16):
                cb = _vgather(cumn, jnp.full((16,), ep, jnp.int32))
                blk_e = blk_e + jnp.where(bvec >= cb, ones, zeros)
            meta_v[pl.ds(half * 16, 16)] = jnp.minimum(blk_e, 15)
        meta_v[pl.ds(32, 16)] = nact
        pltpu.sync_copy(meta_v, meta_hbm)


@functools.partial(
    pl.kernel,
    out_type=[
        jax.ShapeDtypeStruct((T,), jnp.int32),
        jax.ShapeDtypeStruct((48,), jnp.int32),
        jax.ShapeDtypeStruct((TP, D), jnp.float32),
    ],
    mesh=_SC_MESH,
    scratch_types=[
        pltpu.VMEM((TPW,), jnp.int32),
        pltpu.VMEM((TPW,), jnp.int32),
        pltpu.VMEM((NW, 16), jnp.int32),
        pltpu.VMEM((48,), jnp.int32),
        pltpu.VMEM((TPW, D), jnp.float32),
        pltpu.SemaphoreType.DMA,
    ],
)
def _dispatch(idx_hbm, lp_hbm, histin_hbm, x_hbm, pos_hbm, meta_hbm, xs_hbm,
              *scratch):
    _dispatch_body(idx_hbm, lp_hbm, histin_hbm, x_hbm, pos_hbm, meta_hbm,
                   xs_hbm, *scratch)


def _ffn_body(meta_ref, xs_ref, w1_ref, b1_ref, w2_ref, b2_ref, out_ref):
    @pl.when(pl.program_id(0) < meta_ref[32])
    def _():
        x = xs_ref[...]
        h = lax.dot_general(x, w1_ref[0], (((1,), (1,)), ((), ())),
                            preferred_element_type=jnp.float32)
        h = h + b1_ref[0]
        h = 0.5 * h * (1.0 + lax.erf(h * 0.7071067811865476))
        o = lax.dot_general(h, w2_ref[0], (((1,), (1,)), ((), ())),
                            preferred_element_type=jnp.float32)
        out_ref[...] = o + b2_ref[0]


def _ffn(meta, xs, w1, b1, w2, b2):
    grid_spec = pltpu.PrefetchScalarGridSpec(
        num_scalar_prefetch=1,
        grid=(NB,),
        in_specs=[
            pl.BlockSpec((B, D), lambda i, m: (i, 0)),
            pl.BlockSpec((1, H, D), lambda i, m: (m[i], 0, 0)),
            pl.BlockSpec((1, 1, H), lambda i, m: (m[i], 0, 0)),
            pl.BlockSpec((1, D, H), lambda i, m: (m[i], 0, 0)),
            pl.BlockSpec((1, 1, D), lambda i, m: (m[i], 0, 0)),
        ],
        out_specs=pl.BlockSpec((B, D), lambda i, m: (i, 0)),
    )
    return pl.pallas_call(
        _ffn_body,
        grid_spec=grid_spec,
        out_shape=jax.ShapeDtypeStruct((TP, D), jnp.float32),
    )(meta, xs, w1, b1, w2, b2)


def _combine_body(ys_hbm, pos_hbm, out_hbm, pos_v, rows_v, sem):
    cid = lax.axis_index("c")
    sid = lax.axis_index("s")
    w = sid * 2 + cid
    base_t = w * TPW
    pltpu.sync_copy(pos_hbm.at[pl.ds(base_t, TPW)], pos_v)
    pltpu.async_copy(ys_hbm.at[pos_v], rows_v, sem).wait()
    pltpu.sync_copy(rows_v, out_hbm.at[pl.ds(base_t, TPW)])


@functools.partial(
    pl.kernel,
    out_type=jax.ShapeDtypeStruct((T, D), jnp.float32),
    mesh=_SC_MESH,
    scratch_types=[
        pltpu.VMEM((TPW,), jnp.int32),
        pltpu.VMEM((TPW, D), jnp.float32),
        pltpu.SemaphoreType.DMA,
    ],
)
def _combine(ys_hbm, pos_hbm, out_hbm, pos_v, rows_v, sem):
    _combine_body(ys_hbm, pos_hbm, out_hbm, pos_v, rows_v, sem)


def kernel(x, Wr, W1, b1, W2, b2):
    idx2d, loss2d = _router(x, Wr)
    idx = idx2d.reshape(T)
    lp, hist = _count(idx)
    pos, meta, xs = _dispatch(idx, lp, hist, x)
    ys = _ffn(meta, xs, W1, b1.reshape(E, 1, H), W2, b2.reshape(E, 1, D))
    out = _combine(ys, pos)
    return out, loss2d[0, 0]


# speedup vs baseline: 1.4472x; 1.0358x over previous
import functools

import jax
import jax.numpy as jnp
from jax import lax
from jax.experimental import pallas as pl
from jax.experimental.pallas import tpu as pltpu
from jax.experimental.pallas import tpu_sc as plsc

T = 2048
E = 16
D = 768
H = 1536
B = 128
NB = 32
TP = NB * B
TC = 256

NW = 32
TPW = T // NW

_SC_MESH = plsc.VectorSubcoreMesh(
    core_axis_name="c", subcore_axis_name="s", num_cores=2, num_subcores=16)


def _router_body(x_ref, wr_ref, pos_ref, meta_ref, loss_ref):
    x = x_ref[...]
    wr = wr_ref[...]
    logits = lax.dot_general(x, wr, (((1,), (1,)), ((), ())),
                             preferred_element_type=jnp.float32)
    m = jnp.max(logits, axis=-1, keepdims=True)
    eiota = lax.broadcasted_iota(jnp.int32, (T, E), 1)
    idx = jnp.min(jnp.where(logits >= m, eiota, E), axis=-1, keepdims=True)
    p = jnp.exp(logits - m)
    probs = p / jnp.sum(p, axis=-1, keepdims=True)
    onehot = (eiota == idx).astype(jnp.float32)
    su = jnp.sum(probs, axis=0, keepdims=True)
    sc_row = jnp.sum(onehot, axis=0, keepdims=True)
    loss = (float(E) / (float(T) * float(T))) * jnp.sum(su * sc_row)
    loss_ref[...] = loss.reshape(1, 1)

    onesT = jnp.ones((T, 1), jnp.float32)
    scT = lax.dot_general(onehot, onesT, (((0,), (0,)), ((), ())))
    nblkT = jnp.floor((scT + float(B - 1)) * (1.0 / B))
    li = lax.broadcasted_iota(jnp.int32, (E, E), 0)
    lj = lax.broadcasted_iota(jnp.int32, (E, E), 1)
    L16 = (lj <= li).astype(jnp.float32)
    cumnT = lax.dot_general(L16, nblkT, (((1,), (0,)), ((), ())))
    baseT = (cumnT - nblkT) * float(B)

    ci = lax.broadcasted_iota(jnp.int32, (TC, TC), 0)
    cj = lax.broadcasted_iota(jnp.int32, (TC, TC), 1)
    Lc = (cj < ci).astype(jnp.float32)
    prior = jnp.zeros((1, E), jnp.float32)
    ranks = []
    for k in range(T // TC):
        ck = onehot[k * TC:(k + 1) * TC]
        within = lax.dot_general(Lc, ck, (((1,), (0,)), ((), ())))
        rk = jnp.sum(ck * (within + prior), axis=1, keepdims=True)
        ranks.append(rk)
        prior = prior + jnp.sum(ck, axis=0, keepdims=True)
    rank = jnp.concatenate(ranks, axis=0)
    posf = rank + lax.dot_general(onehot, baseT, (((1,), (0,)), ((), ())))
    pos_ref[...] = posf.astype(jnp.int32)

    cumn_b = jnp.broadcast_to(cumnT, (E, 128))
    biota = lax.broadcasted_iota(jnp.int32, (E, 128), 1)
    cmp = (biota >= cumn_b.astype(jnp.int32)).astype(jnp.float32)
    blkrow = jnp.sum(cmp, axis=0, keepdims=True)
    blk = jnp.minimum(blkrow.astype(jnp.int32), E - 1)
    ei0 = lax.broadcasted_iota(jnp.int32, (E, 128), 0)
    nact_row = jnp.sum(jnp.where(ei0 == E - 1, cumn_b, 0.0), axis=0,
                       keepdims=True).astype(jnp.int32)
    liota = lax.broadcasted_iota(jnp.int32, (1, 128), 1)
    metarow = jnp.where(liota < NB, blk,
                        jnp.where(liota == NB, nact_row, 0))
    meta_ref[...] = jnp.broadcast_to(metarow, (8, 128))


def _router(x, wr):
    return pl.pallas_call(
        _router_body,
        out_shape=[
            jax.ShapeDtypeStruct((T, 1), jnp.int32),
            jax.ShapeDtypeStruct((8, 128), jnp.int32),
            jax.ShapeDtypeStruct((1, 1), jnp.float32),
        ],
    )(x, wr)


def _scatter_body(pos_hbm, x_hbm, xs_hbm, pos_v, xrows_v, sem):
    cid = lax.axis_index("c")
    sid = lax.axis_index("s")
    w = sid * 2 + cid
    base_t = w * TPW
    pltpu.sync_copy(pos_hbm.at[pl.ds(base_t, TPW)], pos_v)
    pltpu.sync_copy(x_hbm.at[pl.ds(base_t, TPW)], xrows_v)
    pltpu.async_copy(xrows_v, xs_hbm.at[pos_v], sem).wait()


@functools.partial(
    pl.kernel,
    out_type=jax.ShapeDtypeStruct((TP, D), jnp.float32),
    mesh=_SC_MESH,
    scratch_types=[
        pltpu.VMEM((TPW,), jnp.int32),
        pltpu.VMEM((TPW, D), jnp.float32),
        pltpu.SemaphoreType.DMA,
    ],
)
def _scatter(pos_hbm, x_hbm, xs_hbm, *scratch):
    _scatter_body(pos_hbm, x_hbm, xs_hbm, *scratch)


def _ffn_body(meta_ref, xs_ref, w1_ref, b1_ref, w2_ref, b2_ref, out_ref):
    @pl.when(pl.program_id(0) < meta_ref[NB])
    def _():
        x = xs_ref[...].astype(jnp.bfloat16)
        w1 = w1_ref[0].astype(jnp.bfloat16)
        h = lax.dot_general(x, w1, (((1,), (1,)), ((), ())),
                            preferred_element_type=jnp.float32)
        h = h + b1_ref[0]
        h = 0.5 * h * (1.0 + lax.erf(h * 0.7071067811865476))
        w2 = w2_ref[0].astype(jnp.bfloat16)
        o = lax.dot_general(h.astype(jnp.bfloat16), w2, (((1,), (1,)), ((), ())),
                            preferred_element_type=jnp.float32)
        out_ref[...] = o + b2_ref[0]


def _ffn(meta, xs, w1, b1, w2, b2):
    grid_spec = pltpu.PrefetchScalarGridSpec(
        num_scalar_prefetch=1,
        grid=(NB,),
        in_specs=[
            pl.BlockSpec((B, D), lambda i, m: (i, 0)),
            pl.BlockSpec((1, H, D), lambda i, m: (m[i], 0, 0)),
            pl.BlockSpec((1, 1, H), lambda i, m: (m[i], 0, 0)),
            pl.BlockSpec((1, D, H), lambda i, m: (m[i], 0, 0)),
            pl.BlockSpec((1, 1, D), lambda i, m: (m[i], 0, 0)),
        ],
        out_specs=pl.BlockSpec((B, D), lambda i, m: (i, 0)),
    )
    return pl.pallas_call(
        _ffn_body,
        grid_spec=grid_spec,
        out_shape=jax.ShapeDtypeStruct((TP, D), jnp.float32),
    )(meta, xs, w1, b1, w2, b2)


def _combine_body(ys_hbm, pos_hbm, out_hbm, pos_v, rows_v, sem):
    cid = lax.axis_index("c")
    sid = lax.axis_index("s")
    w = sid * 2 + cid
    base_t = w * TPW
    pltpu.sync_copy(pos_hbm.at[pl.ds(base_t, TPW)], pos_v)
    pltpu.async_copy(ys_hbm.at[pos_v], rows_v, sem).wait()
    pltpu.sync_copy(rows_v, out_hbm.at[pl.ds(base_t, TPW)])


@functools.partial(
    pl.kernel,
    out_type=jax.ShapeDtypeStruct((T, D), jnp.float32),
    mesh=_SC_MESH,
    scratch_types=[
        pltpu.VMEM((TPW,), jnp.int32),
        pltpu.VMEM((TPW, D), jnp.float32),
        pltpu.SemaphoreType.DMA,
    ],
)
def _combine(ys_hbm, pos_hbm, out_hbm, pos_v, rows_v, sem):
    _combine_body(ys_hbm, pos_hbm, out_hbm, pos_v, rows_v, sem)


def kernel(x, Wr, W1, b1, W2, b2):
    pos2d, meta2d, loss2d = _router(x, Wr)
    pos = pos2d.reshape(T)
    meta = meta2d[0]
    xs = _scatter(pos, x)
    ys = _ffn(meta, xs, W1, b1.reshape(E, 1, H), W2, b2.reshape(E, 1, D))
    out = _combine(ys, pos)
    return out, loss2d[0, 0]
